# baseline (speedup 1.0000x reference)
import functools

import jax
import jax.numpy as jnp
from jax import lax
from jax.experimental import pallas as pl
from jax.experimental.pallas import tpu as pltpu
from jax.experimental.pallas import tpu_sc as plsc


def _make_lookup(B, D, NC, NS):
    NW = NC * NS
    b_per_w = B // NW
    mesh = plsc.VectorSubcoreMesh(core_axis_name="c", subcore_axis_name="s")

    @functools.partial(
        pl.kernel,
        mesh=mesh,
        out_type=jax.ShapeDtypeStruct((B, D), jnp.float32),
        scratch_types=[
            pltpu.VMEM((b_per_w,), jnp.int32),
            pltpu.VMEM((b_per_w, D), jnp.float32),
            pltpu.SemaphoreType.DMA,
        ],
        compiler_params=pltpu.CompilerParams(use_tc_tiling_on_sc=False),
    )
    def lookup(idx_hbm, table_hbm, out_hbm, idx_v, rows_v, sem):
        wid = lax.axis_index("s") * NC + lax.axis_index("c")
        base = wid * b_per_w
        pltpu.sync_copy(idx_hbm.at[pl.ds(base, b_per_w)], idx_v)
        pltpu.async_copy(table_hbm.at[idx_v], rows_v, sem).wait()
        pltpu.sync_copy(rows_v, out_hbm.at[pl.ds(base, b_per_w)])

    return lookup


def kernel(sentence, table):
    B = sentence.shape[0]
    D = table.shape[1]
    info = plsc.get_sparse_core_info()
    NC, NS = info.num_cores, info.num_subcores
    lookup = _make_lookup(B, D, NC, NS)
    return lookup(sentence.astype(jnp.int32), table)


# speedup vs baseline: 1.7263x; 1.7263x over previous
import functools

import jax
import jax.numpy as jnp
from jax import lax
from jax.experimental import pallas as pl
from jax.experimental.pallas import tpu as pltpu
from jax.experimental.pallas import tpu_sc as plsc


def _make_lookup(B, D, NC, NS, L):
    NW = NC * NS
    b_per_w = B // NW
    mesh = plsc.VectorSubcoreMesh(core_axis_name="c", subcore_axis_name="s")

    @functools.partial(
        pl.kernel,
        mesh=mesh,
        out_type=jax.ShapeDtypeStruct((B, D), jnp.float32),
        scratch_types=[
            pltpu.VMEM((b_per_w,), jnp.int32),
            pltpu.VMEM((b_per_w, D), jnp.float32),
            pltpu.SemaphoreType.DMA,
        ],
    )
    def lookup(idx_hbm, table_hbm, out_hbm, idx_v, rows_v, sem):
        wid = lax.axis_index("s") * NC + lax.axis_index("c")
        base = wid * b_per_w
        pltpu.sync_copy(idx_hbm.at[pl.ds(base, b_per_w)], idx_v)

        def issue(g, carry):
            vec = idx_v[pl.ds(g * L, L)]
            for j in range(L):
                row = jnp.squeeze(lax.slice(vec, (j,), (j + 1,)))
                pltpu.async_copy(table_hbm.at[row], rows_v.at[g * L + j], sem)
            return carry

        lax.fori_loop(0, b_per_w // L, issue, 0)
        pltpu.make_async_copy(
            table_hbm.at[pl.ds(0, b_per_w)], rows_v, sem
        ).wait()
        pltpu.sync_copy(rows_v, out_hbm.at[pl.ds(base, b_per_w)])

    return lookup


def kernel(sentence, table):
    B = sentence.shape[0]
    D = table.shape[1]
    info = plsc.get_sparse_core_info()
    NC, NS, L = info.num_cores, info.num_subcores, info.num_lanes
    lookup = _make_lookup(B, D, NC, NS, L)
    return lookup(sentence.astype(jnp.int32), table)


# speedup vs baseline: 2.5362x; 1.4692x over previous
import functools

import jax
import jax.numpy as jnp
from jax import lax
from jax.experimental import pallas as pl
from jax.experimental.pallas import tpu as pltpu
from jax.experimental.pallas import tpu_sc as plsc

_CHUNK = 512
_QT = _CHUNK // 128


def _make_lookup(B, V, D, NC, NS, L):
    NW = NC * NS
    n_chunks = (V + _CHUNK - 1) // _CHUNK
    base_cpw = n_chunks // NW
    extra = n_chunks - base_cpw * NW
    n_groups = B // L
    G = D // L
    mesh = plsc.VectorSubcoreMesh(core_axis_name="c", subcore_axis_name="s")

    @functools.partial(
        pl.kernel,
        mesh=mesh,
        out_type=jax.ShapeDtypeStruct((B, D), jnp.float32),
        scratch_types=[
            pltpu.VMEM((B,), jnp.int32),
            pltpu.VMEM((B + L,), jnp.int32),
            pltpu.VMEM((B + L,), jnp.int32),
            pltpu.VMEM((8 * _QT * 8, 128), jnp.float32),
            pltpu.VMEM((2 * L, D), jnp.float32),
            pltpu.SemaphoreType.DMA,
            pltpu.SemaphoreType.DMA,
        ],
        compiler_params=pltpu.CompilerParams(needs_layout_passes=False),
    )
    def lookup(idx_hbm, tab_t_hbm, out_hbm, idx_all, own_j, chk_j, buf,
               stage, tsem, rsem):
        wid = lax.axis_index("s") * NC + lax.axis_index("c")
        pltpu.sync_copy(idx_hbm, idx_all)

        c_start = wid * base_cpw + jnp.minimum(wid, extra)
        c_count = base_cpw + jnp.where(wid < extra, 1, 0)
        lo = c_start * _CHUNK
        hi = (c_start + c_count) * _CHUNK
        iota = lax.iota(jnp.int32, L)
        ebase = [
            8 * _QT * ((16 * g + iota) // 8) + ((16 * g + iota) & 7)
            for g in range(G)
        ]

        def scan_all(p, ptr):
            v = idx_all[pl.ds(p * L, L)]
            m = (v >= lo) & (v < hi)
            plsc.store_compressed(own_j.at[pl.ds(ptr, L)], p * L + iota, mask=m)
            cnt = plsc.all_reduce_population_count(m)
            return ptr + jnp.squeeze(lax.slice(cnt, (0,), (1,)))

        n_own = lax.fori_loop(0, n_groups, scan_all, jnp.int32(0))
        n_own_g = (n_own + L - 1) // L

        def do_chunk(ci, carry):
            c = c_start + ci
            clo = c * _CHUNK

            def scan_own(p, ptr):
                j_raw = own_j[pl.ds(p * L, L)]
                j_c = j_raw & (B - 1)
                v = plsc.load_gather(idx_all, [j_c])
                m = (v >= clo) & (v < clo + _CHUNK) & (p * L + iota < n_own)
                plsc.store_compressed(chk_j.at[pl.ds(ptr, L)], j_c, mask=m)
                cnt = plsc.all_reduce_population_count(m)
                return ptr + jnp.squeeze(lax.slice(cnt, (0,), (1,)))

            n_c = lax.fori_loop(0, n_own_g, scan_own, jnp.int32(0))
            nq = jnp.minimum(_QT, (V - clo + 127) // 128)

            @pl.when(n_c > 0)
            def _():
                for a in range(8):
                    for q in range(_QT):

                        @pl.when(q < nq)
                        def _():
                            pltpu.async_copy(
                                tab_t_hbm.at[pl.ds(8 * a, 8),
                                             pl.ds(clo + 128 * q, 128)],
                                buf.at[pl.ds(8 * (a * _QT + q), 8)],
                                tsem,
                            )

                def tile_drain(i, carry):
                    pltpu.make_async_copy(
                        tab_t_hbm.at[pl.ds(0, 8), pl.ds(0, 128)],
                        buf.at[pl.ds(0, 8)], tsem,
                    ).wait()
                    return carry

                lax.fori_loop(0, 8 * nq, tile_drain, 0)

                n_cg = (n_c + L - 1) // L

                def walk(p, carry):
                    j_vec = chk_j[pl.ds(p * L, L)] & (B - 1)
                    v_vec = plsc.load_gather(idx_all, [j_vec])
                    u_vec = (v_vec - clo) & (_CHUNK - 1)
                    for l in range(L):
                        u = jnp.squeeze(lax.slice(u_vec, (l,), (l + 1,)))
                        j = jnp.squeeze(lax.slice(j_vec, (l,), (l + 1,)))
                        q = u // 128
                        uc = u & 127
                        col_idx = jnp.full((L,), uc, jnp.int32)
                        for g in range(G):
                            row_idx = ebase[g] + 8 * q
                            vals = plsc.load_gather(buf, [row_idx, col_idx])
                            stage[l, pl.ds(16 * g, L)] = vals

                        @pl.when(p * L + l < n_c)
                        def _():
                            pltpu.async_copy(
                                stage.at[l], out_hbm.at[j], rsem
                            )

                    k_p = jnp.minimum(n_c - p * L, L)

                    def row_drain(i, carry):
                        pltpu.make_async_copy(
                            out_hbm.at[pl.ds(0, 1)], stage.at[pl.ds(0, 1)],
                            rsem,
                        ).wait()
                        return carry

                    lax.fori_loop(0, k_p, row_drain, 0)
                    return carry

                lax.fori_loop(0, n_cg, walk, 0)

            return carry

        lax.fori_loop(0, c_count, do_chunk, 0)

    return lookup


def kernel(sentence, table):
    B = sentence.shape[0]
    V, D = table.shape
    info = plsc.get_sparse_core_info()
    NC, NS, L = info.num_cores, info.num_subcores, info.num_lanes
    lookup = _make_lookup(B, V, D, NC, NS, L)
    return lookup(sentence.astype(jnp.int32), table.T)
